# baseline (speedup 1.0000x reference)
import functools

import jax
import jax.numpy as jnp
from jax import lax
from jax.experimental import pallas as pl
from jax.experimental.pallas import tpu as pltpu
from jax.experimental.pallas import tpu_sc as plsc

E = 8
D = 768
H = 2 * D
S = 2048
BLK = 256
P = 6144
NB = P // BLK
BR = 256
NW = 32
TPW = S // NW
LN = 16
DG = D // LN


def _route_kernel(x_ref, Wg_ref, route_ref, counts_ref, acc):
    c = pl.program_id(0)
    xb = x_ref[...]
    logits = lax.dot_general(xb, Wg_ref[...], (((1,), (1,)), ((), ())),
                             preferred_element_type=jnp.float32)
    col = lax.broadcasted_iota(jnp.int32, logits.shape, 1)
    m1 = jnp.max(logits, axis=1, keepdims=True)
    a1 = jnp.min(jnp.where(logits == m1, col, E), axis=1, keepdims=True)
    l2 = jnp.where(col == a1, -jnp.inf, logits)
    m2 = jnp.max(l2, axis=1, keepdims=True)
    a2 = jnp.min(jnp.where(l2 == m2, col, E), axis=1, keepdims=True)
    w1 = 1.0 / (1.0 + jnp.exp(m2 - m1))
    w2 = 1.0 / (1.0 + jnp.exp(m1 - m2))

    M0 = (col == a1).astype(jnp.float32)
    M1 = (col == a2).astype(jnp.float32)
    tri = (lax.broadcasted_iota(jnp.int32, (BR, BR), 0)
           > lax.broadcasted_iota(jnp.int32, (BR, BR), 1)).astype(jnp.float32)
    R0 = lax.dot_general(tri, M0, (((1,), (0,)), ((), ())),
                         preferred_element_type=jnp.float32)
    R1 = lax.dot_general(tri, M1, (((1,), (0,)), ((), ())),
                         preferred_element_type=jnp.float32)

    @pl.when(c == 0)
    def _():
        acc[...] = jnp.zeros_like(acc)

    accv = acc[...]
    cs0 = jnp.sum(M0, axis=0, keepdims=True)
    cs1 = jnp.sum(M1, axis=0, keepdims=True)
    r0 = jnp.sum((R0 + accv) * M0, axis=1, keepdims=True)
    r1 = jnp.sum((R1 + accv + cs0) * M1, axis=1, keepdims=True)
    newacc = accv + cs0 + cs1
    acc[...] = newacc

    route_ref[...] = jnp.concatenate(
        [a1.astype(jnp.float32), a2.astype(jnp.float32), r0, r1, w1, w2,
         jnp.zeros((BR, 2), jnp.float32)], axis=1)
    counts_ref[...] = jnp.concatenate(
        [newacc, jnp.zeros((1, 120), jnp.float32)], axis=1
    ).astype(jnp.int32).reshape(128)


def _route(x2, Wg):
    return pl.pallas_call(
        _route_kernel,
        grid=(S // BR,),
        in_specs=[
            pl.BlockSpec((BR, D), lambda c: (c, 0)),
            pl.BlockSpec((E, D), lambda c: (0, 0)),
        ],
        out_specs=[
            pl.BlockSpec((BR, 8), lambda c: (c, 0)),
            pl.BlockSpec((128,), lambda c: (0,)),
        ],
        out_shape=[
            jax.ShapeDtypeStruct((S, 8), jnp.float32),
            jax.ShapeDtypeStruct((128,), jnp.int32),
        ],
        scratch_shapes=[pltpu.VMEM((1, E), jnp.float32)],
    )(x2, Wg)


def _dyn_take(vec, idx):
    dn = lax.GatherDimensionNumbers(offset_dims=(), collapsed_slice_dims=(0,),
                                    start_index_map=(0,))
    return lax.gather(vec, idx.reshape(16, 1), dn, (1,),
                      mode=lax.GatherScatterMode.PROMISE_IN_BOUNDS)


def _excl_cumsum8(v):
    lane = lax.iota(jnp.int32, 16)
    out = jnp.zeros((16,), v.dtype)
    for k in range(1, 9):
        shifted = _dyn_take(v, jnp.maximum(lane - k, 0))
        out = out + jnp.where(lane >= k, shifted, jnp.zeros((16,), v.dtype))
    return out


@functools.cache
def _make_dispatch():
    mesh = plsc.VectorSubcoreMesh(core_axis_name="c", subcore_axis_name="s")

    @functools.partial(
        pl.kernel, mesh=mesh,
        compiler_params=pltpu.CompilerParams(needs_layout_passes=False),
        out_type=[
            jax.ShapeDtypeStruct((P, D), jnp.float32),
            jax.ShapeDtypeStruct((S,), jnp.int32),
            jax.ShapeDtypeStruct((S,), jnp.int32),
            jax.ShapeDtypeStruct((32,), jnp.int32),
        ],
        scratch_types=[
            pltpu.VMEM((16,), jnp.int32),
            pltpu.VMEM((16,), jnp.int32),
            pltpu.VMEM((TPW * 8,), jnp.float32),
            pltpu.VMEM((TPW,), jnp.int32),
            pltpu.VMEM((TPW,), jnp.int32),
            pltpu.VMEM((TPW, D), jnp.float32),
            pltpu.VMEM((32,), jnp.int32),
            pltpu.SemaphoreType.DMA,
        ],
    )
    def _dispatch(route_hbm, counts_hbm, x_hbm,
                  xg_hbm, pos0_hbm, pos1_hbm, be_hbm,
                  cnt_v, base_v, route_v, p0_v, p1_v, rows_v, be_v, sem):
        wid = lax.axis_index("s") * 2 + lax.axis_index("c")
        tb = wid * TPW

        pltpu.sync_copy(counts_hbm.at[pl.ds(0, 16)], cnt_v)
        cnt = cnt_v[...]
        al = ((cnt + (BLK - 1)) >> 8) << 8
        base_v[...] = _excl_cumsum8(al)

        base_blk = base_v[...] >> 8

        @pl.when(wid == 0)
        def _():
            for g in range(2):
                bv = lax.iota(jnp.int32, 16) + (g * 16)
                bcnt = jnp.full((16,), -1, jnp.int32)
                for e in range(E):
                    start_blk = jnp.broadcast_to(base_blk[e], (16,))
                    bcnt = bcnt + jnp.where(bv >= start_blk,
                                            jnp.int32(1), jnp.int32(0))
                be_v[pl.ds(g * 16, 16)] = bcnt
            pltpu.sync_copy(be_v, be_hbm)

        pltpu.sync_copy(route_hbm.at[pl.ds(tb * 8, TPW * 8)], route_v)
        lane = lax.iota(jnp.int32, 16)
        for g in range(TPW // 16):
            f = (lane + (g * 16)) * 8
            e0 = plsc.load_gather(route_v, [f]).astype(jnp.int32)
            e1 = plsc.load_gather(route_v, [f + 1]).astype(jnp.int32)
            r0 = plsc.load_gather(route_v, [f + 2]).astype(jnp.int32)
            r1 = plsc.load_gather(route_v, [f + 3]).astype(jnp.int32)
            p0_v[pl.ds(g * 16, 16)] = r0 + plsc.load_gather(base_v, [e0])
            p1_v[pl.ds(g * 16, 16)] = r1 + plsc.load_gather(base_v, [e1])

        pltpu.sync_copy(p0_v, pos0_hbm.at[pl.ds(tb, TPW)])
        pltpu.sync_copy(p1_v, pos1_hbm.at[pl.ds(tb, TPW)])

        pltpu.sync_copy(x_hbm.at[pl.ds(tb, TPW)], rows_v)
        pltpu.async_copy(rows_v, xg_hbm.at[p0_v], sem).wait()
        pltpu.async_copy(rows_v, xg_hbm.at[p1_v], sem).wait()

    return _dispatch


def _expert_mm_kernel(be_ref, xg_ref, W1_ref, W2_ref, W3_ref, yg_ref):
    del be_ref
    xb = xg_ref[...].astype(jnp.bfloat16)
    a = lax.dot_general(xb, W1_ref[0], (((1,), (1,)), ((), ())),
                        preferred_element_type=jnp.float32)
    xv = lax.dot_general(xb, W2_ref[0], (((1,), (1,)), ((), ())),
                         preferred_element_type=jnp.float32)
    res = a * (1.0 / (1.0 + jnp.exp(-a))) * xv
    yg_ref[...] = lax.dot_general(res.astype(jnp.bfloat16), W3_ref[0],
                                  (((1,), (1,)), ((), ())),
                                  preferred_element_type=jnp.float32)


def _expert_mm(be, xg, W1, W2, W3):
    grid_spec = pltpu.PrefetchScalarGridSpec(
        num_scalar_prefetch=1,
        grid=(NB,),
        in_specs=[
            pl.BlockSpec((BLK, D), lambda b, be: (b, 0)),
            pl.BlockSpec((1, H, D), lambda b, be: (be[b], 0, 0)),
            pl.BlockSpec((1, H, D), lambda b, be: (be[b], 0, 0)),
            pl.BlockSpec((1, D, H), lambda b, be: (be[b], 0, 0)),
        ],
        out_specs=pl.BlockSpec((BLK, D), lambda b, be: (b, 0)),
    )
    return pl.pallas_call(
        _expert_mm_kernel,
        grid_spec=grid_spec,
        out_shape=jax.ShapeDtypeStruct((P, D), jnp.float32),
    )(be, xg, W1, W2, W3)


def _dyn_bcast(vec, i):
    return _dyn_take(vec, jnp.full((16,), i, jnp.int32))

@functools.cache
def _make_combine():
    mesh = plsc.VectorSubcoreMesh(core_axis_name="c", subcore_axis_name="s")

    @functools.partial(
        pl.kernel, mesh=mesh,
        compiler_params=pltpu.CompilerParams(needs_layout_passes=False),
        out_type=jax.ShapeDtypeStruct((S, D), jnp.float32),
        scratch_types=[
            pltpu.VMEM((TPW,), jnp.int32),
            pltpu.VMEM((TPW,), jnp.int32),
            pltpu.VMEM((TPW * 8,), jnp.float32),
            pltpu.VMEM((TPW, D), jnp.float32),
            pltpu.VMEM((TPW, D), jnp.float32),
            pltpu.SemaphoreType.DMA,
        ],
    )
    def _combine(yg_hbm, route_hbm, pos0_hbm, pos1_hbm, out_hbm,
                 p0_v, p1_v, route_v, rows0_v, rows1_v, sem):
        wid = lax.axis_index("s") * 2 + lax.axis_index("c")
        tb = wid * TPW
        pltpu.sync_copy(pos0_hbm.at[pl.ds(tb, TPW)], p0_v)
        pltpu.sync_copy(pos1_hbm.at[pl.ds(tb, TPW)], p1_v)
        pltpu.async_copy(yg_hbm.at[p0_v], rows0_v, sem).wait()
        pltpu.async_copy(yg_hbm.at[p1_v], rows1_v, sem).wait()
        pltpu.sync_copy(route_hbm.at[pl.ds(tb * 8, TPW * 8)], route_v)

        lane = lax.iota(jnp.int32, 16)
        for g in range(TPW // 16):
            f = (lane + (g * 16)) * 8
            w0g = plsc.load_gather(route_v, [f + 4])
            w1g = plsc.load_gather(route_v, [f + 5])

            def body(i, carry, g=g, w0g=w0g, w1g=w1g):
                t = g * 16 + i
                w0 = _dyn_bcast(w0g, i)
                w1 = _dyn_bcast(w1g, i)
                for j in range(DG):
                    sl = pl.ds(j * LN, LN)
                    rows0_v[t, sl] = rows0_v[t, sl] * w0 + rows1_v[t, sl] * w1
                return carry

            lax.fori_loop(0, 16, body, 0)
        pltpu.sync_copy(rows0_v, out_hbm.at[pl.ds(tb, TPW)])

    return _combine


def kernel(x, Wg, W1, W2, W3):
    x2 = x.reshape(S, D)
    W1 = W1.astype(jnp.bfloat16)
    W2 = W2.astype(jnp.bfloat16)
    W3 = W3.astype(jnp.bfloat16)
    route, counts = _route(x2, Wg)
    routef = route.reshape(S * 8)
    xg, pos0, pos1, be = _make_dispatch()(routef, counts, x2)
    yg = _expert_mm(be, xg, W1, W2, W3)
    out = _make_combine()(yg, routef, pos0, pos1)
    return out.reshape(1, S, D)


# speedup vs baseline: 1.2828x; 1.2828x over previous
import functools

import jax
import jax.numpy as jnp
from jax import lax
from jax.experimental import pallas as pl
from jax.experimental.pallas import tpu as pltpu
from jax.experimental.pallas import tpu_sc as plsc

E = 8
D = 768
H = 2 * D
S = 2048
BLK = 256
P = 6144
NB = P // BLK
BR = 256
NW = 32
TPW = S // NW
LN = 16
DG = D // LN


def _route_kernel(x_ref, Wg_ref, route_ref, counts_ref, acc):
    c = pl.program_id(0)
    xb = x_ref[...]
    logits = lax.dot_general(xb, Wg_ref[...], (((1,), (1,)), ((), ())),
                             preferred_element_type=jnp.float32)
    col = lax.broadcasted_iota(jnp.int32, logits.shape, 1)
    m1 = jnp.max(logits, axis=1, keepdims=True)
    a1 = jnp.min(jnp.where(logits == m1, col, E), axis=1, keepdims=True)
    l2 = jnp.where(col == a1, -jnp.inf, logits)
    m2 = jnp.max(l2, axis=1, keepdims=True)
    a2 = jnp.min(jnp.where(l2 == m2, col, E), axis=1, keepdims=True)
    w1 = 1.0 / (1.0 + jnp.exp(m2 - m1))
    w2 = 1.0 / (1.0 + jnp.exp(m1 - m2))

    M0 = (col == a1).astype(jnp.float32)
    M1 = (col == a2).astype(jnp.float32)
    tri = (lax.broadcasted_iota(jnp.int32, (BR, BR), 0)
           > lax.broadcasted_iota(jnp.int32, (BR, BR), 1)).astype(jnp.float32)
    R0 = lax.dot_general(tri, M0, (((1,), (0,)), ((), ())),
                         preferred_element_type=jnp.float32)
    R1 = lax.dot_general(tri, M1, (((1,), (0,)), ((), ())),
                         preferred_element_type=jnp.float32)

    @pl.when(c == 0)
    def _():
        acc[...] = jnp.zeros_like(acc)

    accv = acc[...]
    cs0 = jnp.sum(M0, axis=0, keepdims=True)
    cs1 = jnp.sum(M1, axis=0, keepdims=True)
    r0 = jnp.sum((R0 + accv) * M0, axis=1, keepdims=True)
    r1 = jnp.sum((R1 + accv + cs0) * M1, axis=1, keepdims=True)
    newacc = accv + cs0 + cs1
    acc[...] = newacc

    route_ref[...] = jnp.concatenate(
        [a1.astype(jnp.float32), a2.astype(jnp.float32), r0, r1, w1, w2,
         jnp.zeros((BR, 2), jnp.float32)], axis=1)
    counts_ref[...] = jnp.concatenate(
        [newacc, jnp.zeros((1, 120), jnp.float32)], axis=1
    ).astype(jnp.int32).reshape(128)


def _route(x2, Wg):
    return pl.pallas_call(
        _route_kernel,
        grid=(S // BR,),
        in_specs=[
            pl.BlockSpec((BR, D), lambda c: (c, 0)),
            pl.BlockSpec((E, D), lambda c: (0, 0)),
        ],
        out_specs=[
            pl.BlockSpec((BR, 8), lambda c: (c, 0)),
            pl.BlockSpec((128,), lambda c: (0,)),
        ],
        out_shape=[
            jax.ShapeDtypeStruct((S, 8), jnp.float32),
            jax.ShapeDtypeStruct((128,), jnp.int32),
        ],
        scratch_shapes=[pltpu.VMEM((1, E), jnp.float32)],
    )(x2, Wg)


def _dyn_take(vec, idx):
    dn = lax.GatherDimensionNumbers(offset_dims=(), collapsed_slice_dims=(0,),
                                    start_index_map=(0,))
    return lax.gather(vec, idx.reshape(16, 1), dn, (1,),
                      mode=lax.GatherScatterMode.PROMISE_IN_BOUNDS)


def _excl_cumsum8(v):
    lane = lax.iota(jnp.int32, 16)
    out = jnp.zeros((16,), v.dtype)
    for k in range(1, 9):
        shifted = _dyn_take(v, jnp.maximum(lane - k, 0))
        out = out + jnp.where(lane >= k, shifted, jnp.zeros((16,), v.dtype))
    return out


@functools.cache
def _make_dispatch():
    mesh = plsc.VectorSubcoreMesh(core_axis_name="c", subcore_axis_name="s")

    @functools.partial(
        pl.kernel, mesh=mesh,
        compiler_params=pltpu.CompilerParams(needs_layout_passes=False),
        out_type=[
            jax.ShapeDtypeStruct((P, D), jnp.float32),
            jax.ShapeDtypeStruct((S,), jnp.int32),
            jax.ShapeDtypeStruct((S,), jnp.int32),
            jax.ShapeDtypeStruct((32,), jnp.int32),
            jax.ShapeDtypeStruct((P, 128), jnp.float32),
        ],
        scratch_types=[
            pltpu.VMEM((16,), jnp.int32),
            pltpu.VMEM((16,), jnp.int32),
            pltpu.VMEM((TPW * 8,), jnp.float32),
            pltpu.VMEM((TPW,), jnp.int32),
            pltpu.VMEM((TPW,), jnp.int32),
            pltpu.VMEM((TPW, D), jnp.float32),
            pltpu.VMEM((32,), jnp.int32),
            pltpu.VMEM((TPW, 128), jnp.float32),
            pltpu.SemaphoreType.DMA,
        ],
    )
    def _dispatch(route_hbm, counts_hbm, x_hbm,
                  xg_hbm, pos0_hbm, pos1_hbm, be_hbm, wpos_hbm,
                  cnt_v, base_v, route_v, p0_v, p1_v, rows_v, be_v, wrows_v,
                  sem):
        wid = lax.axis_index("s") * 2 + lax.axis_index("c")
        tb = wid * TPW

        pltpu.sync_copy(counts_hbm.at[pl.ds(0, 16)], cnt_v)
        cnt = cnt_v[...]
        al = ((cnt + (BLK - 1)) >> 8) << 8
        base_v[...] = _excl_cumsum8(al)

        base_blk = base_v[...] >> 8

        @pl.when(wid == 0)
        def _():
            for g in range(2):
                bv = lax.iota(jnp.int32, 16) + (g * 16)
                bcnt = jnp.full((16,), -1, jnp.int32)
                for e in range(E):
                    start_blk = jnp.broadcast_to(base_blk[e], (16,))
                    bcnt = bcnt + jnp.where(bv >= start_blk,
                                            jnp.int32(1), jnp.int32(0))
                be_v[pl.ds(g * 16, 16)] = bcnt
            pltpu.sync_copy(be_v, be_hbm)

        pltpu.sync_copy(route_hbm.at[pl.ds(tb * 8, TPW * 8)], route_v)
        lane = lax.iota(jnp.int32, 16)
        for g in range(TPW // 16):
            f = (lane + (g * 16)) * 8
            e0 = plsc.load_gather(route_v, [f]).astype(jnp.int32)
            e1 = plsc.load_gather(route_v, [f + 1]).astype(jnp.int32)
            r0 = plsc.load_gather(route_v, [f + 2]).astype(jnp.int32)
            r1 = plsc.load_gather(route_v, [f + 3]).astype(jnp.int32)
            p0_v[pl.ds(g * 16, 16)] = r0 + plsc.load_gather(base_v, [e0])
            p1_v[pl.ds(g * 16, 16)] = r1 + plsc.load_gather(base_v, [e1])

        pltpu.sync_copy(p0_v, pos0_hbm.at[pl.ds(tb, TPW)])
        pltpu.sync_copy(p1_v, pos1_hbm.at[pl.ds(tb, TPW)])

        pltpu.sync_copy(x_hbm.at[pl.ds(tb, TPW)], rows_v)
        pltpu.async_copy(rows_v, xg_hbm.at[p0_v], sem).wait()
        pltpu.async_copy(rows_v, xg_hbm.at[p1_v], sem).wait()

        for slot, pv in ((4, p0_v), (5, p1_v)):
            for g in range(TPW // 16):
                f = (lane + (g * 16)) * 8
                wg = plsc.load_gather(route_v, [f + slot])

                def wbody(i, carry, g=g, wg=wg):
                    wrows_v[g * 16 + i, pl.ds(0, 16)] = _dyn_take(
                        wg, jnp.full((16,), i, jnp.int32))
                    return carry

                lax.fori_loop(0, 16, wbody, 0)
            pltpu.async_copy(wrows_v, wpos_hbm.at[pv], sem).wait()

    return _dispatch


def _expert_mm_kernel(be_ref, xg_ref, wp_ref, W1_ref, W2_ref, W3_ref, yg_ref):
    del be_ref
    xb = xg_ref[...]
    a = lax.dot_general(xb, W1_ref[0], (((1,), (1,)), ((), ())),
                        preferred_element_type=jnp.float32)
    xv = lax.dot_general(xb, W2_ref[0], (((1,), (1,)), ((), ())),
                         preferred_element_type=jnp.float32)
    res = a * (1.0 / (1.0 + jnp.exp(-a))) * xv
    res = res * wp_ref[...][:, :1]
    yg_ref[...] = lax.dot_general(res, W3_ref[0], (((1,), (1,)), ((), ())),
                                  preferred_element_type=jnp.float32)


def _expert_mm(be, xg, wpos, W1, W2, W3):
    grid_spec = pltpu.PrefetchScalarGridSpec(
        num_scalar_prefetch=1,
        grid=(NB,),
        in_specs=[
            pl.BlockSpec((BLK, D), lambda b, be: (b, 0)),
            pl.BlockSpec((BLK, 128), lambda b, be: (b, 0)),
            pl.BlockSpec((1, H, D), lambda b, be: (be[b], 0, 0)),
            pl.BlockSpec((1, H, D), lambda b, be: (be[b], 0, 0)),
            pl.BlockSpec((1, D, H), lambda b, be: (be[b], 0, 0)),
        ],
        out_specs=pl.BlockSpec((BLK, D), lambda b, be: (b, 0)),
    )
    return pl.pallas_call(
        _expert_mm_kernel,
        grid_spec=grid_spec,
        out_shape=jax.ShapeDtypeStruct((P, D), jnp.float32),
    )(be, xg, wpos, W1, W2, W3)


def _dyn_bcast(vec, i):
    return _dyn_take(vec, jnp.full((16,), i, jnp.int32))

@functools.cache
def _make_combine():
    mesh = plsc.VectorSubcoreMesh(core_axis_name="c", subcore_axis_name="s")

    @functools.partial(
        pl.kernel, mesh=mesh,
        compiler_params=pltpu.CompilerParams(needs_layout_passes=False),
        out_type=jax.ShapeDtypeStruct((S, D), jnp.float32),
        scratch_types=[
            pltpu.VMEM((TPW,), jnp.int32),
            pltpu.VMEM((TPW,), jnp.int32),
            pltpu.VMEM((TPW, D), jnp.float32),
            pltpu.VMEM((TPW, D), jnp.float32),
            pltpu.SemaphoreType.DMA,
        ],
    )
    def _combine(yg_hbm, pos0_hbm, pos1_hbm, out_hbm,
                 p0_v, p1_v, rows0_v, rows1_v, sem):
        wid = lax.axis_index("s") * 2 + lax.axis_index("c")
        tb = wid * TPW
        pltpu.sync_copy(pos0_hbm.at[pl.ds(tb, TPW)], p0_v)
        pltpu.sync_copy(pos1_hbm.at[pl.ds(tb, TPW)], p1_v)
        cp0 = pltpu.async_copy(yg_hbm.at[p0_v], rows0_v, sem)
        cp1 = pltpu.async_copy(yg_hbm.at[p1_v], rows1_v, sem)
        cp0.wait()
        cp1.wait()

        def body(t, carry):
            for j in range(DG):
                sl = pl.ds(j * LN, LN)
                rows0_v[t, sl] = rows0_v[t, sl] + rows1_v[t, sl]
            return carry

        lax.fori_loop(0, TPW, body, 0)
        pltpu.sync_copy(rows0_v, out_hbm.at[pl.ds(tb, TPW)])

    return _combine


def kernel(x, Wg, W1, W2, W3):
    x2 = x.reshape(S, D)
    route, counts = _route(x2, Wg)
    routef = route.reshape(S * 8)
    xg, pos0, pos1, be, wpos = _make_dispatch()(routef, counts, x2)
    yg = _expert_mm(be, xg, wpos, W1, W2, W3)
    out = _make_combine()(yg, pos0, pos1)
    return out.reshape(1, S, D)


# speedup vs baseline: 1.5554x; 1.2125x over previous
import functools

import jax
import jax.numpy as jnp
from jax import lax
from jax.experimental import pallas as pl
from jax.experimental.pallas import tpu as pltpu
from jax.experimental.pallas import tpu_sc as plsc

E = 8
D = 768
H = 2 * D
S = 2048
BLK = 256
P = 6144
NB = P // BLK
BR = 256
NW = 32
TPW = S // NW
LN = 16
DG = D // LN


def _route_kernel(x_ref, Wg_ref, route_ref, counts_ref, acc):
    c = pl.program_id(0)
    xb = x_ref[...]
    logits = lax.dot_general(xb, Wg_ref[...], (((1,), (1,)), ((), ())),
                             preferred_element_type=jnp.float32)
    col = lax.broadcasted_iota(jnp.int32, logits.shape, 1)
    m1 = jnp.max(logits, axis=1, keepdims=True)
    a1 = jnp.min(jnp.where(logits == m1, col, E), axis=1, keepdims=True)
    l2 = jnp.where(col == a1, -jnp.inf, logits)
    m2 = jnp.max(l2, axis=1, keepdims=True)
    a2 = jnp.min(jnp.where(l2 == m2, col, E), axis=1, keepdims=True)
    w1 = 1.0 / (1.0 + jnp.exp(m2 - m1))
    w2 = 1.0 / (1.0 + jnp.exp(m1 - m2))

    M0 = (col == a1).astype(jnp.float32)
    M1 = (col == a2).astype(jnp.float32)
    tri = (lax.broadcasted_iota(jnp.int32, (BR, BR), 0)
           > lax.broadcasted_iota(jnp.int32, (BR, BR), 1)).astype(jnp.float32)
    R0 = lax.dot_general(tri, M0, (((1,), (0,)), ((), ())),
                         preferred_element_type=jnp.float32)
    R1 = lax.dot_general(tri, M1, (((1,), (0,)), ((), ())),
                         preferred_element_type=jnp.float32)

    @pl.when(c == 0)
    def _():
        acc[...] = jnp.zeros_like(acc)

    accv = acc[...]
    cs0 = jnp.sum(M0, axis=0, keepdims=True)
    cs1 = jnp.sum(M1, axis=0, keepdims=True)
    r0 = jnp.sum((R0 + accv) * M0, axis=1, keepdims=True)
    r1 = jnp.sum((R1 + accv + cs0) * M1, axis=1, keepdims=True)
    newacc = accv + cs0 + cs1
    acc[...] = newacc

    route_ref[...] = jnp.concatenate(
        [a1.astype(jnp.float32), a2.astype(jnp.float32), r0, r1, w1, w2,
         jnp.zeros((BR, 2), jnp.float32)], axis=1)
    counts_ref[...] = jnp.concatenate(
        [newacc, jnp.zeros((1, 120), jnp.float32)], axis=1
    ).astype(jnp.int32).reshape(128)


def _route(x2, Wg):
    return pl.pallas_call(
        _route_kernel,
        grid=(S // BR,),
        in_specs=[
            pl.BlockSpec((BR, D), lambda c: (c, 0)),
            pl.BlockSpec((E, D), lambda c: (0, 0)),
        ],
        out_specs=[
            pl.BlockSpec((BR, 8), lambda c: (c, 0)),
            pl.BlockSpec((128,), lambda c: (0,)),
        ],
        out_shape=[
            jax.ShapeDtypeStruct((S, 8), jnp.float32),
            jax.ShapeDtypeStruct((128,), jnp.int32),
        ],
        scratch_shapes=[pltpu.VMEM((1, E), jnp.float32)],
    )(x2, Wg)


def _dyn_take(vec, idx):
    dn = lax.GatherDimensionNumbers(offset_dims=(), collapsed_slice_dims=(0,),
                                    start_index_map=(0,))
    return lax.gather(vec, idx.reshape(16, 1), dn, (1,),
                      mode=lax.GatherScatterMode.PROMISE_IN_BOUNDS)


def _excl_cumsum8(v):
    lane = lax.iota(jnp.int32, 16)
    out = jnp.zeros((16,), v.dtype)
    for k in range(1, 9):
        shifted = _dyn_take(v, jnp.maximum(lane - k, 0))
        out = out + jnp.where(lane >= k, shifted, jnp.zeros((16,), v.dtype))
    return out


@functools.cache
def _make_dispatch():
    mesh = plsc.VectorSubcoreMesh(core_axis_name="c", subcore_axis_name="s")

    @functools.partial(
        pl.kernel, mesh=mesh,
        compiler_params=pltpu.CompilerParams(needs_layout_passes=False),
        out_type=[
            jax.ShapeDtypeStruct((P, D), jnp.float32),
            jax.ShapeDtypeStruct((S,), jnp.int32),
            jax.ShapeDtypeStruct((S,), jnp.int32),
            jax.ShapeDtypeStruct((5, 32), jnp.int32),
            jax.ShapeDtypeStruct((P, 128), jnp.float32),
        ],
        scratch_types=[
            pltpu.VMEM((16,), jnp.int32),
            pltpu.VMEM((16,), jnp.int32),
            pltpu.VMEM((TPW * 8,), jnp.float32),
            pltpu.VMEM((TPW,), jnp.int32),
            pltpu.VMEM((TPW,), jnp.int32),
            pltpu.VMEM((TPW, D), jnp.float32),
            pltpu.VMEM((5, 32), jnp.int32),
            pltpu.VMEM((TPW, 128), jnp.float32),
            pltpu.SemaphoreType.DMA,
        ],
    )
    def _dispatch(route_hbm, counts_hbm, x_hbm,
                  xg_hbm, pos0_hbm, pos1_hbm, aux_hbm, wpos_hbm,
                  cnt_v, base_v, route_v, p0_v, p1_v, rows_v, aux_v, wrows_v,
                  sem):
        wid = lax.axis_index("s") * 2 + lax.axis_index("c")
        tb = wid * TPW

        pltpu.sync_copy(counts_hbm.at[pl.ds(0, 16)], cnt_v)
        cnt = cnt_v[...]
        al = ((cnt + (BLK - 1)) >> 8) << 8
        base_v[...] = _excl_cumsum8(al)

        base_blk = base_v[...] >> 8

        @pl.when(wid == 0)
        def _():
            one = jnp.int32(1)
            zero32 = jnp.int32(0)
            lane16 = lax.iota(jnp.int32, 16)
            used = jnp.broadcast_to(base_blk[8], (16,))
            bcnts = []
            for g in range(2):
                bv = lane16 + (g * 16)
                sl = pl.ds(g * 16, 16)
                bcnt = jnp.full((16,), -1, jnp.int32)
                for e in range(E):
                    start_blk = jnp.broadcast_to(base_blk[e], (16,))
                    bcnt = bcnt + jnp.where(bv >= start_blk, one, zero32)
                bcnts.append(bcnt)
                aux_v[0, sl] = bcnt
                pf = jnp.full((16,), E, jnp.int32)
                cnt_lt = jnp.zeros((16,), jnp.int32)
                for e in range(E):
                    nonempty = jnp.broadcast_to(al[e], (16,)) > 0
                    pf = jnp.minimum(
                        pf, jnp.where(nonempty & (bcnt < e),
                                      jnp.full((16,), e, jnp.int32),
                                      jnp.full((16,), E, jnp.int32)))
                    cnt_lt = cnt_lt + jnp.where(nonempty & (bcnt > e), one,
                                                zero32)
                aux_v[2, sl] = pf
                aux_v[3, sl] = jnp.minimum(bv, used - 1)
                aux_v[4, sl] = jnp.bitwise_and(cnt_lt, 1)
            prev0 = _dyn_take(bcnts[0], jnp.maximum(lane16 - 1, 0))
            f0 = jnp.where((lane16 == 0) | (bcnts[0] != prev0), one, zero32)
            last0 = _dyn_take(bcnts[0], jnp.full((16,), 15, jnp.int32))
            prev1 = _dyn_take(bcnts[1], jnp.maximum(lane16 - 1, 0))
            prev1 = jnp.where(lane16 == 0, last0, prev1)
            f1 = jnp.where(bcnts[1] != prev1, one, zero32)
            aux_v[1, pl.ds(0, 16)] = f0
            aux_v[1, pl.ds(16, 16)] = f1
            pltpu.sync_copy(aux_v, aux_hbm)

        pltpu.sync_copy(route_hbm.at[pl.ds(tb * 8, TPW * 8)], route_v)
        lane = lax.iota(jnp.int32, 16)
        for g in range(TPW // 16):
            f = (lane + (g * 16)) * 8
            e0 = plsc.load_gather(route_v, [f]).astype(jnp.int32)
            e1 = plsc.load_gather(route_v, [f + 1]).astype(jnp.int32)
            r0 = plsc.load_gather(route_v, [f + 2]).astype(jnp.int32)
            r1 = plsc.load_gather(route_v, [f + 3]).astype(jnp.int32)
            p0_v[pl.ds(g * 16, 16)] = r0 + plsc.load_gather(base_v, [e0])
            p1_v[pl.ds(g * 16, 16)] = r1 + plsc.load_gather(base_v, [e1])

        pltpu.sync_copy(p0_v, pos0_hbm.at[pl.ds(tb, TPW)])
        pltpu.sync_copy(p1_v, pos1_hbm.at[pl.ds(tb, TPW)])

        pltpu.sync_copy(x_hbm.at[pl.ds(tb, TPW)], rows_v)
        pltpu.async_copy(rows_v, xg_hbm.at[p0_v], sem).wait()
        pltpu.async_copy(rows_v, xg_hbm.at[p1_v], sem).wait()

        for slot, pv in ((4, p0_v), (5, p1_v)):
            for g in range(TPW // 16):
                f = (lane + (g * 16)) * 8
                wg = plsc.load_gather(route_v, [f + slot])

                def wbody(i, carry, g=g, wg=wg):
                    wrows_v[g * 16 + i, pl.ds(0, 16)] = _dyn_take(
                        wg, jnp.full((16,), i, jnp.int32))
                    return carry

                lax.fori_loop(0, 16, wbody, 0)
            pltpu.async_copy(wrows_v, wpos_hbm.at[pv], sem).wait()

    return _dispatch


def _expert_mm_kernel(aux_ref, xg_ref, wp_ref, W1_hbm, W2_hbm, W3_hbm,
                      yg_ref, w1b, w2b, w3b, sems):
    b = pl.program_id(0)
    e = aux_ref[0, b]
    first = aux_ref[1, b]
    pf = aux_ref[2, b]
    active = aux_ref[3, b] == b
    slot = aux_ref[4, b]

    def mk(tens_hbm, buf, dslot, expert):
        return pltpu.make_async_copy(tens_hbm.at[expert], buf.at[dslot],
                                     sems.at[dslot])

    @pl.when(b == 0)
    def _():
        mk(W1_hbm, w1b, slot, e).start()
        mk(W2_hbm, w2b, slot, e).start()
        mk(W3_hbm, w3b, slot, e).start()

    @pl.when(first == 1)
    def _():
        @pl.when(pf < E)
        def _():
            mk(W1_hbm, w1b, 1 - slot, pf).start()
            mk(W2_hbm, w2b, 1 - slot, pf).start()
            mk(W3_hbm, w3b, 1 - slot, pf).start()

        mk(W1_hbm, w1b, slot, e).wait()
        mk(W2_hbm, w2b, slot, e).wait()
        mk(W3_hbm, w3b, slot, e).wait()

    @pl.when(active)
    def _():
        xb = xg_ref[...]
        a = lax.dot_general(xb, w1b[slot], (((1,), (1,)), ((), ())),
                            preferred_element_type=jnp.float32)
        xv = lax.dot_general(xb, w2b[slot], (((1,), (1,)), ((), ())),
                             preferred_element_type=jnp.float32)
        res = a * (1.0 / (1.0 + jnp.exp(-a))) * xv
        res = res * wp_ref[...][:, :1]
        yg_ref[...] = lax.dot_general(res, w3b[slot], (((1,), (1,)), ((), ())),
                                      preferred_element_type=jnp.float32)


def _expert_mm(aux, xg, wpos, W1, W2, W3):
    grid_spec = pltpu.PrefetchScalarGridSpec(
        num_scalar_prefetch=1,
        grid=(NB,),
        in_specs=[
            pl.BlockSpec((BLK, D), lambda b, aux: (aux[3, b], 0)),
            pl.BlockSpec((BLK, 128), lambda b, aux: (aux[3, b], 0)),
            pl.BlockSpec(memory_space=pl.ANY),
            pl.BlockSpec(memory_space=pl.ANY),
            pl.BlockSpec(memory_space=pl.ANY),
        ],
        out_specs=pl.BlockSpec((BLK, D), lambda b, aux: (aux[3, b], 0)),
        scratch_shapes=[
            pltpu.VMEM((2, H, D), jnp.float32),
            pltpu.VMEM((2, H, D), jnp.float32),
            pltpu.VMEM((2, D, H), jnp.float32),
            pltpu.SemaphoreType.DMA((2,)),
        ],
    )
    return pl.pallas_call(
        _expert_mm_kernel,
        grid_spec=grid_spec,
        out_shape=jax.ShapeDtypeStruct((P, D), jnp.float32),
    )(aux, xg, wpos, W1, W2, W3)


def _dyn_bcast(vec, i):
    return _dyn_take(vec, jnp.full((16,), i, jnp.int32))

@functools.cache
def _make_combine():
    mesh = plsc.VectorSubcoreMesh(core_axis_name="c", subcore_axis_name="s")

    @functools.partial(
        pl.kernel, mesh=mesh,
        compiler_params=pltpu.CompilerParams(needs_layout_passes=False),
        out_type=jax.ShapeDtypeStruct((S, D), jnp.float32),
        scratch_types=[
            pltpu.VMEM((TPW,), jnp.int32),
            pltpu.VMEM((TPW,), jnp.int32),
            pltpu.VMEM((TPW, D), jnp.float32),
            pltpu.VMEM((TPW, D), jnp.float32),
            pltpu.SemaphoreType.DMA,
        ],
    )
    def _combine(yg_hbm, pos0_hbm, pos1_hbm, out_hbm,
                 p0_v, p1_v, rows0_v, rows1_v, sem):
        wid = lax.axis_index("s") * 2 + lax.axis_index("c")
        tb = wid * TPW
        pltpu.sync_copy(pos0_hbm.at[pl.ds(tb, TPW)], p0_v)
        pltpu.sync_copy(pos1_hbm.at[pl.ds(tb, TPW)], p1_v)
        cp0 = pltpu.async_copy(yg_hbm.at[p0_v], rows0_v, sem)
        cp1 = pltpu.async_copy(yg_hbm.at[p1_v], rows1_v, sem)
        cp0.wait()
        cp1.wait()

        def body(t, carry):
            for j in range(DG):
                sl = pl.ds(j * LN, LN)
                rows0_v[t, sl] = rows0_v[t, sl] + rows1_v[t, sl]
            return carry

        lax.fori_loop(0, TPW, body, 0)
        pltpu.sync_copy(rows0_v, out_hbm.at[pl.ds(tb, TPW)])

    return _combine


def kernel(x, Wg, W1, W2, W3):
    x2 = x.reshape(S, D)
    route, counts = _route(x2, Wg)
    routef = route.reshape(S * 8)
    xg, pos0, pos1, aux, wpos = _make_dispatch()(routef, counts, x2)
    yg = _expert_mm(aux, xg, wpos, W1, W2, W3)
    out = _make_combine()(yg, pos0, pos1)
    return out.reshape(1, S, D)


# speedup vs baseline: 1.5871x; 1.0204x over previous
import functools

import jax
import jax.numpy as jnp
from jax import lax
from jax.experimental import pallas as pl
from jax.experimental.pallas import tpu as pltpu
from jax.experimental.pallas import tpu_sc as plsc

E = 8
D = 768
H = 2 * D
S = 2048
BLK = 256
P = 6144
NB = P // BLK
BR = 512
NW = 32
TPW = S // NW
LN = 16
DG = D // LN


def _route_kernel(x_ref, Wg_ref, route_ref, counts_ref, acc):
    c = pl.program_id(0)
    xb = x_ref[...]
    logits = lax.dot_general(xb, Wg_ref[...], (((1,), (1,)), ((), ())),
                             preferred_element_type=jnp.float32)
    col = lax.broadcasted_iota(jnp.int32, logits.shape, 1)
    m1 = jnp.max(logits, axis=1, keepdims=True)
    a1 = jnp.min(jnp.where(logits == m1, col, E), axis=1, keepdims=True)
    l2 = jnp.where(col == a1, -jnp.inf, logits)
    m2 = jnp.max(l2, axis=1, keepdims=True)
    a2 = jnp.min(jnp.where(l2 == m2, col, E), axis=1, keepdims=True)
    w1 = 1.0 / (1.0 + jnp.exp(m2 - m1))
    w2 = 1.0 / (1.0 + jnp.exp(m1 - m2))

    M0 = (col == a1).astype(jnp.float32)
    M1 = (col == a2).astype(jnp.float32)
    tri = (lax.broadcasted_iota(jnp.int32, (BR, BR), 0)
           > lax.broadcasted_iota(jnp.int32, (BR, BR), 1)).astype(jnp.bfloat16)
    R0 = lax.dot_general(tri, M0.astype(jnp.bfloat16),
                         (((1,), (0,)), ((), ())),
                         preferred_element_type=jnp.float32)
    R1 = lax.dot_general(tri, M1.astype(jnp.bfloat16),
                         (((1,), (0,)), ((), ())),
                         preferred_element_type=jnp.float32)

    @pl.when(c == 0)
    def _():
        acc[...] = jnp.zeros_like(acc)

    accv = acc[...]
    cs0 = jnp.sum(M0, axis=0, keepdims=True)
    cs1 = jnp.sum(M1, axis=0, keepdims=True)
    r0 = jnp.sum((R0 + accv) * M0, axis=1, keepdims=True)
    r1 = jnp.sum((R1 + accv + cs0) * M1, axis=1, keepdims=True)
    newacc = accv + cs0 + cs1
    acc[...] = newacc

    route_ref[...] = jnp.concatenate(
        [a1.astype(jnp.float32), a2.astype(jnp.float32), r0, r1, w1, w2,
         jnp.zeros((BR, 2), jnp.float32)], axis=1)
    counts_ref[...] = jnp.concatenate(
        [newacc, jnp.zeros((1, 120), jnp.float32)], axis=1
    ).astype(jnp.int32).reshape(128)


def _route(x2, Wg):
    return pl.pallas_call(
        _route_kernel,
        grid=(S // BR,),
        in_specs=[
            pl.BlockSpec((BR, D), lambda c: (c, 0)),
            pl.BlockSpec((E, D), lambda c: (0, 0)),
        ],
        out_specs=[
            pl.BlockSpec((BR, 8), lambda c: (c, 0)),
            pl.BlockSpec((128,), lambda c: (0,)),
        ],
        out_shape=[
            jax.ShapeDtypeStruct((S, 8), jnp.float32),
            jax.ShapeDtypeStruct((128,), jnp.int32),
        ],
        scratch_shapes=[pltpu.VMEM((1, E), jnp.float32)],
    )(x2, Wg)


def _dyn_take(vec, idx):
    dn = lax.GatherDimensionNumbers(offset_dims=(), collapsed_slice_dims=(0,),
                                    start_index_map=(0,))
    return lax.gather(vec, idx.reshape(16, 1), dn, (1,),
                      mode=lax.GatherScatterMode.PROMISE_IN_BOUNDS)


def _excl_cumsum8(v):
    lane = lax.iota(jnp.int32, 16)
    out = jnp.zeros((16,), v.dtype)
    for k in range(1, 9):
        shifted = _dyn_take(v, jnp.maximum(lane - k, 0))
        out = out + jnp.where(lane >= k, shifted, jnp.zeros((16,), v.dtype))
    return out


@functools.cache
def _make_dispatch():
    mesh = plsc.VectorSubcoreMesh(core_axis_name="c", subcore_axis_name="s")

    @functools.partial(
        pl.kernel, mesh=mesh,
        compiler_params=pltpu.CompilerParams(needs_layout_passes=False),
        out_type=[
            jax.ShapeDtypeStruct((P, D), jnp.float32),
            jax.ShapeDtypeStruct((S,), jnp.int32),
            jax.ShapeDtypeStruct((S,), jnp.int32),
            jax.ShapeDtypeStruct((5, 32), jnp.int32),
            jax.ShapeDtypeStruct((P, 128), jnp.float32),
        ],
        scratch_types=[
            pltpu.VMEM((16,), jnp.int32),
            pltpu.VMEM((16,), jnp.int32),
            pltpu.VMEM((TPW * 8,), jnp.float32),
            pltpu.VMEM((TPW,), jnp.int32),
            pltpu.VMEM((TPW,), jnp.int32),
            pltpu.VMEM((TPW, D), jnp.float32),
            pltpu.VMEM((5, 32), jnp.int32),
            pltpu.VMEM((TPW, 128), jnp.float32),
            pltpu.SemaphoreType.DMA,
        ],
    )
    def _dispatch(route_hbm, counts_hbm, x_hbm,
                  xg_hbm, pos0_hbm, pos1_hbm, aux_hbm, wpos_hbm,
                  cnt_v, base_v, route_v, p0_v, p1_v, rows_v, aux_v, wrows_v,
                  sem):
        wid = lax.axis_index("s") * 2 + lax.axis_index("c")
        tb = wid * TPW

        pltpu.sync_copy(counts_hbm.at[pl.ds(0, 16)], cnt_v)
        cnt = cnt_v[...]
        al = ((cnt + (BLK - 1)) >> 8) << 8
        base_v[...] = _excl_cumsum8(al)

        base_blk = base_v[...] >> 8

        @pl.when(wid == 0)
        def _():
            one = jnp.int32(1)
            zero32 = jnp.int32(0)
            lane16 = lax.iota(jnp.int32, 16)
            used = jnp.broadcast_to(base_blk[8], (16,))
            bcnts = []
            for g in range(2):
                bv = lane16 + (g * 16)
                sl = pl.ds(g * 16, 16)
                bcnt = jnp.full((16,), -1, jnp.int32)
                for e in range(E):
                    start_blk = jnp.broadcast_to(base_blk[e], (16,))
                    bcnt = bcnt + jnp.where(bv >= start_blk, one, zero32)
                bcnts.append(bcnt)
                aux_v[0, sl] = bcnt
                pf = jnp.full((16,), E, jnp.int32)
                cnt_lt = jnp.zeros((16,), jnp.int32)
                for e in range(E):
                    nonempty = jnp.broadcast_to(al[e], (16,)) > 0
                    pf = jnp.minimum(
                        pf, jnp.where(nonempty & (bcnt < e),
                                      jnp.full((16,), e, jnp.int32),
                                      jnp.full((16,), E, jnp.int32)))
                    cnt_lt = cnt_lt + jnp.where(nonempty & (bcnt > e), one,
                                                zero32)
                aux_v[2, sl] = pf
                aux_v[3, sl] = jnp.minimum(bv, used - 1)
                aux_v[4, sl] = jnp.bitwise_and(cnt_lt, 1)
            prev0 = _dyn_take(bcnts[0], jnp.maximum(lane16 - 1, 0))
            f0 = jnp.where((lane16 == 0) | (bcnts[0] != prev0), one, zero32)
            last0 = _dyn_take(bcnts[0], jnp.full((16,), 15, jnp.int32))
            prev1 = _dyn_take(bcnts[1], jnp.maximum(lane16 - 1, 0))
            prev1 = jnp.where(lane16 == 0, last0, prev1)
            f1 = jnp.where(bcnts[1] != prev1, one, zero32)
            aux_v[1, pl.ds(0, 16)] = f0
            aux_v[1, pl.ds(16, 16)] = f1
            pltpu.sync_copy(aux_v, aux_hbm)

        pltpu.sync_copy(route_hbm.at[pl.ds(tb * 8, TPW * 8)], route_v)
        lane = lax.iota(jnp.int32, 16)
        for g in range(TPW // 16):
            f = (lane + (g * 16)) * 8
            e0 = plsc.load_gather(route_v, [f]).astype(jnp.int32)
            e1 = plsc.load_gather(route_v, [f + 1]).astype(jnp.int32)
            r0 = plsc.load_gather(route_v, [f + 2]).astype(jnp.int32)
            r1 = plsc.load_gather(route_v, [f + 3]).astype(jnp.int32)
            p0_v[pl.ds(g * 16, 16)] = r0 + plsc.load_gather(base_v, [e0])
            p1_v[pl.ds(g * 16, 16)] = r1 + plsc.load_gather(base_v, [e1])

        pltpu.sync_copy(p0_v, pos0_hbm.at[pl.ds(tb, TPW)])
        pltpu.sync_copy(p1_v, pos1_hbm.at[pl.ds(tb, TPW)])

        pltpu.sync_copy(x_hbm.at[pl.ds(tb, TPW)], rows_v)
        pltpu.async_copy(rows_v, xg_hbm.at[p0_v], sem).wait()
        pltpu.async_copy(rows_v, xg_hbm.at[p1_v], sem).wait()

        for slot, pv in ((4, p0_v), (5, p1_v)):
            for g in range(TPW // 16):
                f = (lane + (g * 16)) * 8
                wg = plsc.load_gather(route_v, [f + slot])

                def wbody(i, carry, g=g, wg=wg):
                    wrows_v[g * 16 + i, pl.ds(0, 16)] = _dyn_take(
                        wg, jnp.full((16,), i, jnp.int32))
                    return carry

                lax.fori_loop(0, 16, wbody, 0)
            pltpu.async_copy(wrows_v, wpos_hbm.at[pv], sem).wait()

    return _dispatch


def _expert_mm_kernel(aux_ref, xg_ref, wp_ref, W1_hbm, W2_hbm, W3_hbm,
                      yg_ref, w1b, w2b, w3b, sems):
    b = pl.program_id(0)
    e = aux_ref[0, b]
    first = aux_ref[1, b]
    pf = aux_ref[2, b]
    active = aux_ref[3, b] == b
    slot = aux_ref[4, b]

    def mk(tens_hbm, buf, dslot, expert):
        return pltpu.make_async_copy(tens_hbm.at[expert], buf.at[dslot],
                                     sems.at[dslot])

    @pl.when(b == 0)
    def _():
        mk(W1_hbm, w1b, slot, e).start()
        mk(W2_hbm, w2b, slot, e).start()
        mk(W3_hbm, w3b, slot, e).start()

    @pl.when(first == 1)
    def _():
        @pl.when(pf < E)
        def _():
            mk(W1_hbm, w1b, 1 - slot, pf).start()
            mk(W2_hbm, w2b, 1 - slot, pf).start()
            mk(W3_hbm, w3b, 1 - slot, pf).start()

        mk(W1_hbm, w1b, slot, e).wait()
        mk(W2_hbm, w2b, slot, e).wait()
        mk(W3_hbm, w3b, slot, e).wait()

    @pl.when(active)
    def _():
        xb = xg_ref[...]
        a = lax.dot_general(xb, w1b[slot], (((1,), (1,)), ((), ())),
                            preferred_element_type=jnp.float32)
        xv = lax.dot_general(xb, w2b[slot], (((1,), (1,)), ((), ())),
                             preferred_element_type=jnp.float32)
        res = a * (1.0 / (1.0 + jnp.exp(-a))) * xv
        res = res * wp_ref[...][:, :1]
        yg_ref[...] = lax.dot_general(res, w3b[slot], (((1,), (1,)), ((), ())),
                                      preferred_element_type=jnp.float32)


def _expert_mm(aux, xg, wpos, W1, W2, W3):
    grid_spec = pltpu.PrefetchScalarGridSpec(
        num_scalar_prefetch=1,
        grid=(NB,),
        in_specs=[
            pl.BlockSpec((BLK, D), lambda b, aux: (aux[3, b], 0)),
            pl.BlockSpec((BLK, 128), lambda b, aux: (aux[3, b], 0)),
            pl.BlockSpec(memory_space=pl.ANY),
            pl.BlockSpec(memory_space=pl.ANY),
            pl.BlockSpec(memory_space=pl.ANY),
        ],
        out_specs=pl.BlockSpec((BLK, D), lambda b, aux: (aux[3, b], 0)),
        scratch_shapes=[
            pltpu.VMEM((2, H, D), jnp.float32),
            pltpu.VMEM((2, H, D), jnp.float32),
            pltpu.VMEM((2, D, H), jnp.float32),
            pltpu.SemaphoreType.DMA((2,)),
        ],
    )
    return pl.pallas_call(
        _expert_mm_kernel,
        grid_spec=grid_spec,
        out_shape=jax.ShapeDtypeStruct((P, D), jnp.float32),
    )(aux, xg, wpos, W1, W2, W3)


def _dyn_bcast(vec, i):
    return _dyn_take(vec, jnp.full((16,), i, jnp.int32))

@functools.cache
def _make_combine():
    mesh = plsc.VectorSubcoreMesh(core_axis_name="c", subcore_axis_name="s")

    @functools.partial(
        pl.kernel, mesh=mesh,
        compiler_params=pltpu.CompilerParams(needs_layout_passes=False),
        out_type=jax.ShapeDtypeStruct((S, D), jnp.float32),
        scratch_types=[
            pltpu.VMEM((TPW,), jnp.int32),
            pltpu.VMEM((TPW,), jnp.int32),
            pltpu.VMEM((TPW, D), jnp.float32),
            pltpu.VMEM((TPW, D), jnp.float32),
            pltpu.SemaphoreType.DMA,
        ],
    )
    def _combine(yg_hbm, pos0_hbm, pos1_hbm, out_hbm,
                 p0_v, p1_v, rows0_v, rows1_v, sem):
        wid = lax.axis_index("s") * 2 + lax.axis_index("c")
        tb = wid * TPW
        pltpu.sync_copy(pos0_hbm.at[pl.ds(tb, TPW)], p0_v)
        pltpu.sync_copy(pos1_hbm.at[pl.ds(tb, TPW)], p1_v)
        cp0 = pltpu.async_copy(yg_hbm.at[p0_v], rows0_v, sem)
        cp1 = pltpu.async_copy(yg_hbm.at[p1_v], rows1_v, sem)
        cp0.wait()
        cp1.wait()

        def body(t, carry):
            for j in range(DG):
                sl = pl.ds(j * LN, LN)
                rows0_v[t, sl] = rows0_v[t, sl] + rows1_v[t, sl]
            return carry

        lax.fori_loop(0, TPW, body, 0)
        pltpu.sync_copy(rows0_v, out_hbm.at[pl.ds(tb, TPW)])

    return _combine


def kernel(x, Wg, W1, W2, W3):
    x2 = x.reshape(S, D)
    route, counts = _route(x2, Wg)
    routef = route.reshape(S * 8)
    xg, pos0, pos1, aux, wpos = _make_dispatch()(routef, counts, x2)
    yg = _expert_mm(aux, xg, wpos, W1, W2, W3)
    out = _make_combine()(yg, pos0, pos1)
    return out.reshape(1, S, D)


# speedup vs baseline: 1.6101x; 1.0145x over previous
import functools

import jax
import jax.numpy as jnp
from jax import lax
from jax.experimental import pallas as pl
from jax.experimental.pallas import tpu as pltpu
from jax.experimental.pallas import tpu_sc as plsc

E = 8
D = 768
H = 2 * D
S = 2048
BLK = 256
P = 6144
NB = P // BLK
BR = 512
NW = 32
TPW = S // NW
LN = 16
DG = D // LN


def _route_kernel(x_ref, Wg_ref, route_ref, counts_ref, acc):
    c = pl.program_id(0)
    xb = x_ref[...]
    logits = lax.dot_general(xb, Wg_ref[...], (((1,), (1,)), ((), ())),
                             preferred_element_type=jnp.float32)
    col = lax.broadcasted_iota(jnp.int32, logits.shape, 1)
    m1 = jnp.max(logits, axis=1, keepdims=True)
    a1 = jnp.min(jnp.where(logits == m1, col, E), axis=1, keepdims=True)
    l2 = jnp.where(col == a1, -jnp.inf, logits)
    m2 = jnp.max(l2, axis=1, keepdims=True)
    a2 = jnp.min(jnp.where(l2 == m2, col, E), axis=1, keepdims=True)
    w1 = 1.0 / (1.0 + jnp.exp(m2 - m1))
    w2 = 1.0 / (1.0 + jnp.exp(m1 - m2))

    M0 = (col == a1).astype(jnp.float32)
    M1 = (col == a2).astype(jnp.float32)
    tri = (lax.broadcasted_iota(jnp.int32, (BR, BR), 0)
           > lax.broadcasted_iota(jnp.int32, (BR, BR), 1)).astype(jnp.bfloat16)
    R0 = lax.dot_general(tri, M0.astype(jnp.bfloat16),
                         (((1,), (0,)), ((), ())),
                         preferred_element_type=jnp.float32)
    R1 = lax.dot_general(tri, M1.astype(jnp.bfloat16),
                         (((1,), (0,)), ((), ())),
                         preferred_element_type=jnp.float32)

    @pl.when(c == 0)
    def _():
        acc[...] = jnp.zeros_like(acc)

    accv = acc[...]
    cs0 = jnp.sum(M0, axis=0, keepdims=True)
    cs1 = jnp.sum(M1, axis=0, keepdims=True)
    r0 = jnp.sum((R0 + accv) * M0, axis=1, keepdims=True)
    r1 = jnp.sum((R1 + accv + cs0) * M1, axis=1, keepdims=True)
    newacc = accv + cs0 + cs1
    acc[...] = newacc

    route_ref[...] = jnp.concatenate(
        [a1.astype(jnp.float32), a2.astype(jnp.float32), r0, r1, w1, w2,
         jnp.zeros((BR, 2), jnp.float32)], axis=1)
    counts_ref[...] = jnp.concatenate(
        [newacc, jnp.zeros((1, 120), jnp.float32)], axis=1
    ).astype(jnp.int32).reshape(128)


def _route(x2, Wg):
    return pl.pallas_call(
        _route_kernel,
        grid=(S // BR,),
        in_specs=[
            pl.BlockSpec((BR, D), lambda c: (c, 0)),
            pl.BlockSpec((E, D), lambda c: (0, 0)),
        ],
        out_specs=[
            pl.BlockSpec((BR, 8), lambda c: (c, 0)),
            pl.BlockSpec((128,), lambda c: (0,)),
        ],
        out_shape=[
            jax.ShapeDtypeStruct((S, 8), jnp.float32),
            jax.ShapeDtypeStruct((128,), jnp.int32),
        ],
        scratch_shapes=[pltpu.VMEM((1, E), jnp.float32)],
    )(x2, Wg)


def _dyn_take(vec, idx):
    dn = lax.GatherDimensionNumbers(offset_dims=(), collapsed_slice_dims=(0,),
                                    start_index_map=(0,))
    return lax.gather(vec, idx.reshape(16, 1), dn, (1,),
                      mode=lax.GatherScatterMode.PROMISE_IN_BOUNDS)


def _excl_cumsum8(v):
    lane = lax.iota(jnp.int32, 16)
    out = jnp.zeros((16,), v.dtype)
    for k in range(1, 9):
        shifted = _dyn_take(v, jnp.maximum(lane - k, 0))
        out = out + jnp.where(lane >= k, shifted, jnp.zeros((16,), v.dtype))
    return out


@functools.cache
def _make_dispatch():
    mesh = plsc.VectorSubcoreMesh(core_axis_name="c", subcore_axis_name="s")

    @functools.partial(
        pl.kernel, mesh=mesh,
        compiler_params=pltpu.CompilerParams(needs_layout_passes=False),
        out_type=[
            jax.ShapeDtypeStruct((P, D), jnp.float32),
            jax.ShapeDtypeStruct((S,), jnp.int32),
            jax.ShapeDtypeStruct((S,), jnp.int32),
            jax.ShapeDtypeStruct((5, 32), jnp.int32),
            jax.ShapeDtypeStruct((P, 128), jnp.float32),
        ],
        scratch_types=[
            pltpu.VMEM((16,), jnp.int32),
            pltpu.VMEM((16,), jnp.int32),
            pltpu.VMEM((TPW, 8), jnp.float32),
            pltpu.VMEM((TPW,), jnp.int32),
            pltpu.VMEM((TPW,), jnp.int32),
            pltpu.VMEM((TPW, D), jnp.float32),
            pltpu.VMEM((5, 32), jnp.int32),
            pltpu.VMEM((TPW, 128), jnp.float32),
            pltpu.SemaphoreType.DMA,
        ],
    )
    def _dispatch(route_hbm, counts_hbm, x_hbm,
                  xg_hbm, pos0_hbm, pos1_hbm, aux_hbm, wpos_hbm,
                  cnt_v, base_v, route_v, p0_v, p1_v, rows_v, aux_v, wrows_v,
                  sem):
        wid = lax.axis_index("s") * 2 + lax.axis_index("c")
        tb = wid * TPW

        pltpu.sync_copy(counts_hbm.at[pl.ds(0, 16)], cnt_v)
        cnt = cnt_v[...]
        al = ((cnt + (BLK - 1)) >> 8) << 8
        base_v[...] = _excl_cumsum8(al)

        base_blk = base_v[...] >> 8

        @pl.when(wid == 0)
        def _():
            one = jnp.int32(1)
            zero32 = jnp.int32(0)
            lane16 = lax.iota(jnp.int32, 16)
            used = jnp.broadcast_to(base_blk[8], (16,))
            bcnts = []
            for g in range(2):
                bv = lane16 + (g * 16)
                sl = pl.ds(g * 16, 16)
                bcnt = jnp.full((16,), -1, jnp.int32)
                for e in range(E):
                    start_blk = jnp.broadcast_to(base_blk[e], (16,))
                    bcnt = bcnt + jnp.where(bv >= start_blk, one, zero32)
                bcnts.append(bcnt)
                aux_v[0, sl] = bcnt
                pf = jnp.full((16,), E, jnp.int32)
                cnt_lt = jnp.zeros((16,), jnp.int32)
                for e in range(E):
                    nonempty = jnp.broadcast_to(al[e], (16,)) > 0
                    pf = jnp.minimum(
                        pf, jnp.where(nonempty & (bcnt < e),
                                      jnp.full((16,), e, jnp.int32),
                                      jnp.full((16,), E, jnp.int32)))
                    cnt_lt = cnt_lt + jnp.where(nonempty & (bcnt > e), one,
                                                zero32)
                aux_v[2, sl] = pf
                aux_v[3, sl] = jnp.minimum(bv, used - 1)
                aux_v[4, sl] = jnp.bitwise_and(cnt_lt, 1)
            prev0 = _dyn_take(bcnts[0], jnp.maximum(lane16 - 1, 0))
            f0 = jnp.where((lane16 == 0) | (bcnts[0] != prev0), one, zero32)
            last0 = _dyn_take(bcnts[0], jnp.full((16,), 15, jnp.int32))
            prev1 = _dyn_take(bcnts[1], jnp.maximum(lane16 - 1, 0))
            prev1 = jnp.where(lane16 == 0, last0, prev1)
            f1 = jnp.where(bcnts[1] != prev1, one, zero32)
            aux_v[1, pl.ds(0, 16)] = f0
            aux_v[1, pl.ds(16, 16)] = f1
            pltpu.sync_copy(aux_v, aux_hbm)

        pltpu.sync_copy(route_hbm.at[pl.ds(tb, TPW)], route_v)
        lane = lax.iota(jnp.int32, 16)
        zero = jnp.zeros((16,), jnp.int32)
        for g in range(TPW // 16):
            rows = lane + (g * 16)
            e0 = plsc.load_gather(route_v, [rows, zero]).astype(jnp.int32)
            e1 = plsc.load_gather(route_v, [rows, zero + 1]).astype(jnp.int32)
            r0 = plsc.load_gather(route_v, [rows, zero + 2]).astype(jnp.int32)
            r1 = plsc.load_gather(route_v, [rows, zero + 3]).astype(jnp.int32)
            p0_v[pl.ds(g * 16, 16)] = r0 + plsc.load_gather(base_v, [e0])
            p1_v[pl.ds(g * 16, 16)] = r1 + plsc.load_gather(base_v, [e1])

        pltpu.sync_copy(p0_v, pos0_hbm.at[pl.ds(tb, TPW)])
        pltpu.sync_copy(p1_v, pos1_hbm.at[pl.ds(tb, TPW)])

        pltpu.sync_copy(x_hbm.at[pl.ds(tb, TPW)], rows_v)
        pltpu.async_copy(rows_v, xg_hbm.at[p0_v], sem).wait()
        pltpu.async_copy(rows_v, xg_hbm.at[p1_v], sem).wait()

        for slot, pv in ((4, p0_v), (5, p1_v)):
            for g in range(TPW // 16):
                rows = lane + (g * 16)
                wg = plsc.load_gather(route_v, [rows, zero + slot])

                def wbody(i, carry, g=g, wg=wg):
                    wrows_v[g * 16 + i, pl.ds(0, 16)] = _dyn_take(
                        wg, jnp.full((16,), i, jnp.int32))
                    return carry

                lax.fori_loop(0, 16, wbody, 0)
            pltpu.async_copy(wrows_v, wpos_hbm.at[pv], sem).wait()

    return _dispatch


def _expert_mm_kernel(aux_ref, xg_ref, wp_ref, W1_hbm, W2_hbm, W3_hbm,
                      yg_ref, w1b, w2b, w3b, sems):
    b = pl.program_id(0)
    e = aux_ref[0, b]
    first = aux_ref[1, b]
    pf = aux_ref[2, b]
    active = aux_ref[3, b] == b
    slot = aux_ref[4, b]

    def mk(tens_hbm, buf, dslot, expert):
        return pltpu.make_async_copy(tens_hbm.at[expert], buf.at[dslot],
                                     sems.at[dslot])

    @pl.when(b == 0)
    def _():
        mk(W1_hbm, w1b, slot, e).start()
        mk(W2_hbm, w2b, slot, e).start()
        mk(W3_hbm, w3b, slot, e).start()

    @pl.when(first == 1)
    def _():
        @pl.when(pf < E)
        def _():
            mk(W1_hbm, w1b, 1 - slot, pf).start()
            mk(W2_hbm, w2b, 1 - slot, pf).start()
            mk(W3_hbm, w3b, 1 - slot, pf).start()

        mk(W1_hbm, w1b, slot, e).wait()
        mk(W2_hbm, w2b, slot, e).wait()
        mk(W3_hbm, w3b, slot, e).wait()

    @pl.when(active)
    def _():
        xb = xg_ref[...]
        a = lax.dot_general(xb, w1b[slot], (((1,), (1,)), ((), ())),
                            preferred_element_type=jnp.float32)
        xv = lax.dot_general(xb, w2b[slot], (((1,), (1,)), ((), ())),
                             preferred_element_type=jnp.float32)
        res = a * (1.0 / (1.0 + jnp.exp(-a))) * xv
        res = res * wp_ref[...][:, :1]
        yg_ref[...] = lax.dot_general(res, w3b[slot], (((1,), (1,)), ((), ())),
                                      preferred_element_type=jnp.float32)


def _expert_mm(aux, xg, wpos, W1, W2, W3):
    grid_spec = pltpu.PrefetchScalarGridSpec(
        num_scalar_prefetch=1,
        grid=(NB,),
        in_specs=[
            pl.BlockSpec((BLK, D), lambda b, aux: (aux[3, b], 0)),
            pl.BlockSpec((BLK, 128), lambda b, aux: (aux[3, b], 0)),
            pl.BlockSpec(memory_space=pl.ANY),
            pl.BlockSpec(memory_space=pl.ANY),
            pl.BlockSpec(memory_space=pl.ANY),
        ],
        out_specs=pl.BlockSpec((BLK, D), lambda b, aux: (aux[3, b], 0)),
        scratch_shapes=[
            pltpu.VMEM((2, H, D), jnp.float32),
            pltpu.VMEM((2, H, D), jnp.float32),
            pltpu.VMEM((2, D, H), jnp.float32),
            pltpu.SemaphoreType.DMA((2,)),
        ],
    )
    return pl.pallas_call(
        _expert_mm_kernel,
        grid_spec=grid_spec,
        out_shape=jax.ShapeDtypeStruct((P, D), jnp.float32),
    )(aux, xg, wpos, W1, W2, W3)


def _dyn_bcast(vec, i):
    return _dyn_take(vec, jnp.full((16,), i, jnp.int32))

@functools.cache
def _make_combine():
    mesh = plsc.VectorSubcoreMesh(core_axis_name="c", subcore_axis_name="s")

    @functools.partial(
        pl.kernel, mesh=mesh,
        compiler_params=pltpu.CompilerParams(needs_layout_passes=False),
        out_type=jax.ShapeDtypeStruct((S, D), jnp.float32),
        scratch_types=[
            pltpu.VMEM((TPW,), jnp.int32),
            pltpu.VMEM((TPW,), jnp.int32),
            pltpu.VMEM((TPW, D), jnp.float32),
            pltpu.VMEM((TPW, D), jnp.float32),
            pltpu.SemaphoreType.DMA,
        ],
    )
    def _combine(yg_hbm, pos0_hbm, pos1_hbm, out_hbm,
                 p0_v, p1_v, rows0_v, rows1_v, sem):
        wid = lax.axis_index("s") * 2 + lax.axis_index("c")
        tb = wid * TPW
        pltpu.sync_copy(pos0_hbm.at[pl.ds(tb, TPW)], p0_v)
        pltpu.sync_copy(pos1_hbm.at[pl.ds(tb, TPW)], p1_v)
        cp0 = pltpu.async_copy(yg_hbm.at[p0_v], rows0_v, sem)
        cp1 = pltpu.async_copy(yg_hbm.at[p1_v], rows1_v, sem)
        cp0.wait()
        cp1.wait()

        def body(t, carry):
            for j in range(DG):
                sl = pl.ds(j * LN, LN)
                rows0_v[t, sl] = rows0_v[t, sl] + rows1_v[t, sl]
            return carry

        lax.fori_loop(0, TPW, body, 0)
        pltpu.sync_copy(rows0_v, out_hbm.at[pl.ds(tb, TPW)])

    return _combine


def kernel(x, Wg, W1, W2, W3):
    x2 = x.reshape(S, D)
    route, counts = _route(x2, Wg)
    xg, pos0, pos1, aux, wpos = _make_dispatch()(route, counts, x2)
    yg = _expert_mm(aux, xg, wpos, W1, W2, W3)
    out = _make_combine()(yg, pos0, pos1)
    return out.reshape(1, S, D)


# speedup vs baseline: 1.6103x; 1.0002x over previous
import functools

import jax
import jax.numpy as jnp
from jax import lax
from jax.experimental import pallas as pl
from jax.experimental.pallas import tpu as pltpu
from jax.experimental.pallas import tpu_sc as plsc

E = 8
D = 768
H = 2 * D
S = 2048
BLK = 256
P = 6144
NB = P // BLK
BR = 512
NW = 32
TPW = S // NW
LN = 16
DG = D // LN


def _route_kernel(x_ref, Wg_ref, route_ref, counts_ref, acc):
    c = pl.program_id(0)
    xb = x_ref[...]
    logits = lax.dot_general(xb, Wg_ref[...], (((1,), (1,)), ((), ())),
                             preferred_element_type=jnp.float32)
    col = lax.broadcasted_iota(jnp.int32, logits.shape, 1)
    m1 = jnp.max(logits, axis=1, keepdims=True)
    a1 = jnp.min(jnp.where(logits == m1, col, E), axis=1, keepdims=True)
    l2 = jnp.where(col == a1, -jnp.inf, logits)
    m2 = jnp.max(l2, axis=1, keepdims=True)
    a2 = jnp.min(jnp.where(l2 == m2, col, E), axis=1, keepdims=True)
    w1 = 1.0 / (1.0 + jnp.exp(m2 - m1))
    w2 = 1.0 / (1.0 + jnp.exp(m1 - m2))

    M0 = (col == a1).astype(jnp.float32)
    M1 = (col == a2).astype(jnp.float32)
    tri = (lax.broadcasted_iota(jnp.int32, (BR, BR), 0)
           > lax.broadcasted_iota(jnp.int32, (BR, BR), 1)).astype(jnp.bfloat16)
    R0 = lax.dot_general(tri, M0.astype(jnp.bfloat16),
                         (((1,), (0,)), ((), ())),
                         preferred_element_type=jnp.float32)
    R1 = lax.dot_general(tri, M1.astype(jnp.bfloat16),
                         (((1,), (0,)), ((), ())),
                         preferred_element_type=jnp.float32)

    @pl.when(c == 0)
    def _():
        acc[...] = jnp.zeros_like(acc)

    accv = acc[...]
    cs0 = jnp.sum(M0, axis=0, keepdims=True)
    cs1 = jnp.sum(M1, axis=0, keepdims=True)
    r0 = jnp.sum((R0 + accv) * M0, axis=1, keepdims=True)
    r1 = jnp.sum((R1 + accv + cs0) * M1, axis=1, keepdims=True)
    newacc = accv + cs0 + cs1
    acc[...] = newacc

    route_ref[...] = jnp.concatenate(
        [a1.astype(jnp.float32), a2.astype(jnp.float32), r0, r1, w1, w2,
         jnp.zeros((BR, 2), jnp.float32)], axis=1)
    counts_ref[...] = jnp.concatenate(
        [newacc, jnp.zeros((1, 120), jnp.float32)], axis=1
    ).astype(jnp.int32).reshape(128)


def _route(x2, Wg):
    return pl.pallas_call(
        _route_kernel,
        grid=(S // BR,),
        in_specs=[
            pl.BlockSpec((BR, D), lambda c: (c, 0)),
            pl.BlockSpec((E, D), lambda c: (0, 0)),
        ],
        out_specs=[
            pl.BlockSpec((BR, 8), lambda c: (c, 0)),
            pl.BlockSpec((128,), lambda c: (0,)),
        ],
        out_shape=[
            jax.ShapeDtypeStruct((S, 8), jnp.float32),
            jax.ShapeDtypeStruct((128,), jnp.int32),
        ],
        scratch_shapes=[pltpu.VMEM((1, E), jnp.float32)],
    )(x2, Wg)


def _dyn_take(vec, idx):
    dn = lax.GatherDimensionNumbers(offset_dims=(), collapsed_slice_dims=(0,),
                                    start_index_map=(0,))
    return lax.gather(vec, idx.reshape(16, 1), dn, (1,),
                      mode=lax.GatherScatterMode.PROMISE_IN_BOUNDS)


def _excl_cumsum8(v):
    lane = lax.iota(jnp.int32, 16)
    out = jnp.zeros((16,), v.dtype)
    for k in range(1, 9):
        shifted = _dyn_take(v, jnp.maximum(lane - k, 0))
        out = out + jnp.where(lane >= k, shifted, jnp.zeros((16,), v.dtype))
    return out


@functools.cache
def _make_dispatch():
    mesh = plsc.VectorSubcoreMesh(core_axis_name="c", subcore_axis_name="s")

    @functools.partial(
        pl.kernel, mesh=mesh,
        compiler_params=pltpu.CompilerParams(needs_layout_passes=False),
        out_type=[
            jax.ShapeDtypeStruct((P, D), jnp.float32),
            jax.ShapeDtypeStruct((S,), jnp.int32),
            jax.ShapeDtypeStruct((S,), jnp.int32),
            jax.ShapeDtypeStruct((5, 32), jnp.int32),
            jax.ShapeDtypeStruct((P, 128), jnp.float32),
        ],
        scratch_types=[
            pltpu.VMEM((16,), jnp.int32),
            pltpu.VMEM((16,), jnp.int32),
            pltpu.VMEM((TPW, 8), jnp.float32),
            pltpu.VMEM((TPW,), jnp.int32),
            pltpu.VMEM((TPW,), jnp.int32),
            pltpu.VMEM((TPW, D), jnp.float32),
            pltpu.VMEM((5, 32), jnp.int32),
            pltpu.VMEM((TPW, 128), jnp.float32),
            pltpu.SemaphoreType.DMA,
        ],
    )
    def _dispatch(route_hbm, counts_hbm, x_hbm,
                  xg_hbm, pos0_hbm, pos1_hbm, aux_hbm, wpos_hbm,
                  cnt_v, base_v, route_v, p0_v, p1_v, rows_v, aux_v, wrows_v,
                  sem):
        wid = lax.axis_index("s") * 2 + lax.axis_index("c")
        tb = wid * TPW

        pltpu.sync_copy(counts_hbm.at[pl.ds(0, 16)], cnt_v)
        cnt = cnt_v[...]
        al = ((cnt + (BLK - 1)) >> 8) << 8
        base_v[...] = _excl_cumsum8(al)

        base_blk = base_v[...] >> 8

        @pl.when(wid == 0)
        def _():
            one = jnp.int32(1)
            zero32 = jnp.int32(0)
            lane16 = lax.iota(jnp.int32, 16)
            used = jnp.broadcast_to(base_blk[8], (16,))
            bcnts = []
            for g in range(2):
                bv = lane16 + (g * 16)
                sl = pl.ds(g * 16, 16)
                bcnt = jnp.full((16,), -1, jnp.int32)
                for e in range(E):
                    start_blk = jnp.broadcast_to(base_blk[e], (16,))
                    bcnt = bcnt + jnp.where(bv >= start_blk, one, zero32)
                bcnts.append(bcnt)
                aux_v[0, sl] = bcnt
                pf = jnp.full((16,), E, jnp.int32)
                cnt_lt = jnp.zeros((16,), jnp.int32)
                for e in range(E):
                    nonempty = jnp.broadcast_to(al[e], (16,)) > 0
                    pf = jnp.minimum(
                        pf, jnp.where(nonempty & (bcnt < e),
                                      jnp.full((16,), e, jnp.int32),
                                      jnp.full((16,), E, jnp.int32)))
                    cnt_lt = cnt_lt + jnp.where(nonempty & (bcnt > e), one,
                                                zero32)
                aux_v[2, sl] = pf
                aux_v[3, sl] = jnp.minimum(bv, used - 1)
                aux_v[4, sl] = jnp.bitwise_and(cnt_lt, 1)
            prev0 = _dyn_take(bcnts[0], jnp.maximum(lane16 - 1, 0))
            f0 = jnp.where((lane16 == 0) | (bcnts[0] != prev0), one, zero32)
            last0 = _dyn_take(bcnts[0], jnp.full((16,), 15, jnp.int32))
            prev1 = _dyn_take(bcnts[1], jnp.maximum(lane16 - 1, 0))
            prev1 = jnp.where(lane16 == 0, last0, prev1)
            f1 = jnp.where(bcnts[1] != prev1, one, zero32)
            aux_v[1, pl.ds(0, 16)] = f0
            aux_v[1, pl.ds(16, 16)] = f1
            pltpu.sync_copy(aux_v, aux_hbm)

        pltpu.sync_copy(route_hbm.at[pl.ds(tb, TPW)], route_v)
        lane = lax.iota(jnp.int32, 16)
        zero = jnp.zeros((16,), jnp.int32)
        for g in range(TPW // 16):
            rows = lane + (g * 16)
            e0 = plsc.load_gather(route_v, [rows, zero]).astype(jnp.int32)
            e1 = plsc.load_gather(route_v, [rows, zero + 1]).astype(jnp.int32)
            r0 = plsc.load_gather(route_v, [rows, zero + 2]).astype(jnp.int32)
            r1 = plsc.load_gather(route_v, [rows, zero + 3]).astype(jnp.int32)
            p0_v[pl.ds(g * 16, 16)] = r0 + plsc.load_gather(base_v, [e0])
            p1_v[pl.ds(g * 16, 16)] = r1 + plsc.load_gather(base_v, [e1])

        pltpu.sync_copy(p0_v, pos0_hbm.at[pl.ds(tb, TPW)])
        pltpu.sync_copy(p1_v, pos1_hbm.at[pl.ds(tb, TPW)])

        pltpu.sync_copy(x_hbm.at[pl.ds(tb, TPW)], rows_v)
        pltpu.async_copy(rows_v, xg_hbm.at[p0_v], sem).wait()
        pltpu.async_copy(rows_v, xg_hbm.at[p1_v], sem).wait()

        for slot, pv in ((4, p0_v), (5, p1_v)):
            for g in range(TPW // 16):
                rows = lane + (g * 16)
                wg = plsc.load_gather(route_v, [rows, zero + slot])

                def wbody(i, carry, g=g, wg=wg):
                    wrows_v[g * 16 + i, pl.ds(0, 16)] = _dyn_take(
                        wg, jnp.full((16,), i, jnp.int32))
                    return carry

                lax.fori_loop(0, 16, wbody, 0)
            pltpu.async_copy(wrows_v, wpos_hbm.at[pv], sem).wait()

    return _dispatch


def _expert_mm_kernel(aux_ref, xg_ref, wp_ref, W1_hbm, W2_hbm, W3_hbm,
                      yg_ref, w1b, w2b, w3b, sems):
    b = pl.program_id(0)
    e = aux_ref[0, b]
    first = aux_ref[1, b]
    pf = aux_ref[2, b]
    active = aux_ref[3, b] == b
    slot = aux_ref[4, b]

    def mk(tens_hbm, buf, dslot, expert):
        return pltpu.make_async_copy(tens_hbm.at[expert], buf.at[dslot],
                                     sems.at[dslot])

    @pl.when(b == 0)
    def _():
        mk(W1_hbm, w1b, slot, e).start()
        mk(W2_hbm, w2b, slot, e).start()
        mk(W3_hbm, w3b, slot, e).start()

    @pl.when(first == 1)
    def _():
        @pl.when(pf < E)
        def _():
            mk(W1_hbm, w1b, 1 - slot, pf).start()
            mk(W2_hbm, w2b, 1 - slot, pf).start()
            mk(W3_hbm, w3b, 1 - slot, pf).start()

        mk(W1_hbm, w1b, slot, e).wait()
        mk(W2_hbm, w2b, slot, e).wait()
        mk(W3_hbm, w3b, slot, e).wait()

    @pl.when(active)
    def _():
        xb = xg_ref[...]
        a = lax.dot_general(xb, w1b[slot], (((1,), (1,)), ((), ())),
                            preferred_element_type=jnp.float32)
        xv = lax.dot_general(xb, w2b[slot], (((1,), (1,)), ((), ())),
                             preferred_element_type=jnp.float32)
        res = a * (1.0 / (1.0 + jnp.exp(-a))) * xv
        res = res * wp_ref[...][:, :1]
        yg_ref[...] = lax.dot_general(res, w3b[slot], (((1,), (1,)), ((), ())),
                                      preferred_element_type=jnp.float32)


def _expert_mm(aux, xg, wpos, W1, W2, W3):
    grid_spec = pltpu.PrefetchScalarGridSpec(
        num_scalar_prefetch=1,
        grid=(NB,),
        in_specs=[
            pl.BlockSpec((BLK, D), lambda b, aux: (aux[3, b], 0)),
            pl.BlockSpec((BLK, 128), lambda b, aux: (aux[3, b], 0)),
            pl.BlockSpec(memory_space=pl.ANY),
            pl.BlockSpec(memory_space=pl.ANY),
            pl.BlockSpec(memory_space=pl.ANY),
        ],
        out_specs=pl.BlockSpec((BLK, D), lambda b, aux: (aux[3, b], 0)),
        scratch_shapes=[
            pltpu.VMEM((2, H, D), jnp.float32),
            pltpu.VMEM((2, H, D), jnp.float32),
            pltpu.VMEM((2, D, H), jnp.float32),
            pltpu.SemaphoreType.DMA((2,)),
        ],
    )
    return pl.pallas_call(
        _expert_mm_kernel,
        grid_spec=grid_spec,
        out_shape=jax.ShapeDtypeStruct((P, D), jnp.float32),
    )(aux, xg, wpos, W1, W2, W3)


@functools.cache
def _make_combine():
    mesh = plsc.VectorSubcoreMesh(core_axis_name="c", subcore_axis_name="s")

    @functools.partial(
        pl.kernel, mesh=mesh,
        compiler_params=pltpu.CompilerParams(needs_layout_passes=False),
        out_type=jax.ShapeDtypeStruct((S, D), jnp.float32),
        scratch_types=[
            pltpu.VMEM((TPW,), jnp.int32),
            pltpu.VMEM((TPW,), jnp.int32),
            pltpu.VMEM((TPW, D), jnp.float32),
            pltpu.VMEM((TPW, D), jnp.float32),
            pltpu.SemaphoreType.DMA,
        ],
    )
    def _combine(yg_hbm, pos0_hbm, pos1_hbm, out_hbm,
                 p0_v, p1_v, rows0_v, rows1_v, sem):
        wid = lax.axis_index("s") * 2 + lax.axis_index("c")
        tb = wid * TPW
        pltpu.sync_copy(pos0_hbm.at[pl.ds(tb, TPW)], p0_v)
        pltpu.sync_copy(pos1_hbm.at[pl.ds(tb, TPW)], p1_v)
        cp0 = pltpu.async_copy(yg_hbm.at[p0_v], rows0_v, sem)
        cp1 = pltpu.async_copy(yg_hbm.at[p1_v], rows1_v, sem)
        cp0.wait()
        cp1.wait()

        def body(t, carry):
            for j in range(DG):
                sl = pl.ds(j * LN, LN)
                rows0_v[t, sl] = rows0_v[t, sl] + rows1_v[t, sl]
            return carry

        lax.fori_loop(0, TPW, body, 0)
        pltpu.sync_copy(rows0_v, out_hbm.at[pl.ds(tb, TPW)])

    return _combine


def kernel(x, Wg, W1, W2, W3):
    x2 = x.reshape(S, D)
    route, counts = _route(x2, Wg)
    xg, pos0, pos1, aux, wpos = _make_dispatch()(route, counts, x2)
    yg = _expert_mm(aux, xg, wpos, W1, W2, W3)
    out = _make_combine()(yg, pos0, pos1)
    return out.reshape(1, S, D)
